# baseline (speedup 1.0000x reference)
import functools

import jax
import jax.numpy as jnp
import numpy as np
from jax import lax
from jax.experimental import pallas as pl
from jax.experimental.pallas import tpu as pltpu
from jax.experimental.pallas import tpu_sc as plsc

T = 2048
D = 768
H = 12
DH = D // H
E = 8
FF = 4 * D

BM = 128
NB = 2 * T // BM + E
M = NB * BM
FB = 3072
FT = FF // FB

_NEG = float(np.finfo(np.float32).min)


def _qkv_body(x_ref, g_ref, b_ref, w_ref, q_ref, k_ref, v_ref):
    x = x_ref[...]
    m = jnp.mean(x, axis=-1, keepdims=True)
    v = jnp.mean((x - m) * (x - m), axis=-1, keepdims=True)
    h = (x - m) * jax.lax.rsqrt(v + 1e-5) * g_ref[...] + b_ref[...]
    qkv = jnp.dot(h, w_ref[...], preferred_element_type=jnp.float32)
    for hh in range(H):
        q_ref[hh] = qkv[:, hh * DH:(hh + 1) * DH]
        k_ref[hh] = qkv[:, D + hh * DH:D + (hh + 1) * DH]
        v_ref[hh] = qkv[:, 2 * D + hh * DH:2 * D + (hh + 1) * DH]


def _qkv(x, g, b, wq, wk, wv):
    blk = 256
    grid = (T // blk,)
    wqkv = jnp.concatenate([wq, wk, wv], axis=1)
    spec_x = pl.BlockSpec((blk, D), lambda i: (i, 0))
    spec_w = pl.BlockSpec((D, 3 * D), lambda i: (0, 0))
    spec_v = pl.BlockSpec((1, D), lambda i: (0, 0))
    spec_h = pl.BlockSpec((H, blk, DH), lambda i: (0, i, 0))
    return pl.pallas_call(
        _qkv_body,
        grid=grid,
        in_specs=[spec_x, spec_v, spec_v, spec_w],
        out_specs=[spec_h, spec_h, spec_h],
        out_shape=[jax.ShapeDtypeStruct((H, T, DH), jnp.float32)] * 3,
    )(x, g.reshape(1, D), b.reshape(1, D), wqkv)


def _attn_body(q_ref, k_ref, v_ref, o_ref, *, blk):
    i = pl.program_id(1)
    q = q_ref[0]
    k = k_ref[0]
    s = jax.lax.dot_general(q, k, (((1,), (1,)), ((), ())),
                            preferred_element_type=jnp.float32)
    s = s * (1.0 / np.sqrt(DH))
    rows = jax.lax.broadcasted_iota(jnp.int32, (blk, T), 0) + i * blk
    cols = jax.lax.broadcasted_iota(jnp.int32, (blk, T), 1)
    s = jnp.where(cols <= rows, s, _NEG)
    m = jnp.max(s, axis=-1, keepdims=True)
    p = jnp.exp(s - m)
    l = jnp.sum(p, axis=-1, keepdims=True)
    o = jnp.dot(p, v_ref[0], preferred_element_type=jnp.float32)
    o_ref[0] = o / l


def _attn(q, k, v):
    blk = 1024
    grid = (H, T // blk)
    return pl.pallas_call(
        functools.partial(_attn_body, blk=blk),
        grid=grid,
        in_specs=[
            pl.BlockSpec((1, blk, DH), lambda h, i: (h, i, 0)),
            pl.BlockSpec((1, T, DH), lambda h, i: (h, 0, 0)),
            pl.BlockSpec((1, T, DH), lambda h, i: (h, 0, 0)),
        ],
        out_specs=pl.BlockSpec((1, blk, DH), lambda h, i: (h, i, 0)),
        out_shape=jax.ShapeDtypeStruct((H, T, DH), jnp.float32),
    )(q, k, v)


def _proj_body(ctx_ref, hid_ref, wo_ref, g_ref, b_ref, wr_ref,
               x_ref, h2_ref, lg_ref):
    ctx = jnp.concatenate([ctx_ref[hh] for hh in range(H)], axis=-1)
    x = hid_ref[...] + jnp.dot(ctx, wo_ref[...],
                               preferred_element_type=jnp.float32)
    x_ref[...] = x
    m = jnp.mean(x, axis=-1, keepdims=True)
    v = jnp.mean((x - m) * (x - m), axis=-1, keepdims=True)
    h2 = (x - m) * jax.lax.rsqrt(v + 1e-5) * g_ref[...] + b_ref[...]
    h2_ref[...] = h2
    lg_ref[...] = jnp.dot(h2, wr_ref[...], preferred_element_type=jnp.float32)


def _proj(ctx, hid, wo, g, b, wr):
    blk = 256
    grid = (T // blk,)
    spec_x = pl.BlockSpec((blk, D), lambda i: (i, 0))
    return pl.pallas_call(
        _proj_body,
        grid=grid,
        in_specs=[pl.BlockSpec((H, blk, DH), lambda i: (0, i, 0)), spec_x,
                  pl.BlockSpec((D, D), lambda i: (0, 0)),
                  pl.BlockSpec((1, D), lambda i: (0, 0)),
                  pl.BlockSpec((1, D), lambda i: (0, 0)),
                  pl.BlockSpec((D, E), lambda i: (0, 0))],
        out_specs=[spec_x, spec_x, pl.BlockSpec((blk, E), lambda i: (i, 0))],
        out_shape=[jax.ShapeDtypeStruct((T, D), jnp.float32),
                   jax.ShapeDtypeStruct((T, D), jnp.float32),
                   jax.ShapeDtypeStruct((T, E), jnp.float32)],
    )(ctx, hid, wo, g.reshape(1, D), b.reshape(1, D), wr)


def _cumsum0(x):
    sh = 1
    while sh < T:
        x = x + jnp.concatenate(
            [jnp.zeros((sh, E), x.dtype), x[:-sh, :]], axis=0)
        sh *= 2
    return x


def _router_body(lg_ref, d0_ref, d1_ref, w0_ref, w1_ref,
                 gmap_ref, valid_ref, loss_ref):
    lg = lg_ref[...]
    eio = jax.lax.broadcasted_iota(jnp.int32, (T, E), 1)

    v0 = jnp.max(lg, axis=-1, keepdims=True)
    i0 = jnp.min(jnp.where(lg == v0, eio, E), axis=-1, keepdims=True)
    lg1 = jnp.where(eio == i0, _NEG, lg)
    v1 = jnp.max(lg1, axis=-1, keepdims=True)
    i1 = jnp.min(jnp.where(lg1 == v1, eio, E), axis=-1, keepdims=True)

    e1v = jnp.exp(v1 - v0)
    den = 1.0 + e1v
    w0 = 1.0 / den
    w1 = e1v / den
    w0_ref[...] = jnp.broadcast_to(w0, (T, 128))
    w1_ref[...] = jnp.broadcast_to(w1, (T, 128))

    p = jnp.exp(lg - v0)
    probs = p / jnp.sum(p, axis=-1, keepdims=True)
    usage = jnp.sum(probs, axis=0) * (1.0 / T)
    loss_ref[0, 0] = E * jnp.sum(usage * usage)

    oh0 = (eio == i0).astype(jnp.float32)
    oh1 = (eio == i1).astype(jnp.float32)
    c0in = _cumsum0(oh0)
    c1in = _cumsum0(oh1)
    rank0 = jnp.sum(oh0 * (c0in - oh0), axis=-1)
    rank1 = jnp.sum(oh1 * (c1in - oh1), axis=-1)
    cnt0 = c0in[T - 1, :]
    cnt1 = c1in[T - 1, :]

    off = jnp.float32(0.0)
    offs = []
    ends = []
    for e in range(E):
        ce = cnt0[e] + cnt1[e]
        nb_e = jnp.ceil(ce * (1.0 / BM))
        offs.append(off)
        end = off + nb_e * BM
        ends.append(end)
        off = end

    base0 = jnp.zeros((T,), jnp.float32)
    base1 = jnp.zeros((T,), jnp.float32)
    for e in range(E):
        base0 = base0 + oh0[:, e] * offs[e]
        base1 = base1 + oh1[:, e] * (offs[e] + cnt0[e])
    d0_ref[...] = (base0 + rank0).astype(jnp.int32)[None, :]
    d1_ref[...] = (base1 + rank1).astype(jnp.int32)[None, :]

    for b in range(NB):
        g = jnp.int32(0)
        for e in range(E - 1):
            g = g + (ends[e] <= b * BM).astype(jnp.int32)
        gmap_ref[b] = g
        valid_ref[b] = (b * BM < off).astype(jnp.int32)


def _router(lg):
    vm = pl.BlockSpec(memory_space=pltpu.VMEM)
    sm = pl.BlockSpec(memory_space=pltpu.SMEM)
    return pl.pallas_call(
        _router_body,
        in_specs=[vm],
        out_specs=[vm, vm, vm, vm, sm, sm, sm],
        out_shape=[jax.ShapeDtypeStruct((1, T), jnp.int32),
                   jax.ShapeDtypeStruct((1, T), jnp.int32),
                   jax.ShapeDtypeStruct((T, 128), jnp.float32),
                   jax.ShapeDtypeStruct((T, 128), jnp.float32),
                   jax.ShapeDtypeStruct((NB,), jnp.int32),
                   jax.ShapeDtypeStruct((NB,), jnp.int32),
                   jax.ShapeDtypeStruct((1, 1), jnp.float32)],
    )(lg)


_NW = 32
_PPW = 2 * T // _NW


def _dispatch_sc_body(src_hbm, dst_hbm, h2_hbm, out_hbm,
                      src_v, dst_v, rows_v, sem):
    wid = lax.axis_index("s") * 2 + lax.axis_index("c")
    base = wid * _PPW
    pltpu.sync_copy(src_hbm.at[pl.ds(base, _PPW)], src_v)
    pltpu.sync_copy(dst_hbm.at[pl.ds(base, _PPW)], dst_v)
    pltpu.async_copy(h2_hbm.at[src_v], rows_v, sem).wait()
    pltpu.async_copy(rows_v, out_hbm.at[dst_v], sem).wait()


def _dispatch(h2, src, dst):
    mesh = plsc.VectorSubcoreMesh(core_axis_name="c", subcore_axis_name="s")
    return pl.kernel(
        _dispatch_sc_body,
        mesh=mesh,
        out_type=jax.ShapeDtypeStruct((M, D), jnp.float32),
        scratch_types=[
            pltpu.VMEM((_PPW,), jnp.int32),
            pltpu.VMEM((_PPW,), jnp.int32),
            pltpu.VMEM((_PPW, D), jnp.float32),
            pltpu.SemaphoreType.DMA,
        ],
    )(src, dst, h2)


def _moe_body(gmap_ref, valid_ref, xs_ref, w1_ref, w3_ref, w2_ref, y_ref):
    b = pl.program_id(0)

    @pl.when(valid_ref[b] == 1)
    def _():
        x = xs_ref[...]
        h1 = jnp.dot(x, w1_ref[0], preferred_element_type=jnp.float32)
        h3 = jnp.dot(x, w3_ref[0], preferred_element_type=jnp.float32)
        act = h1 / (1.0 + jnp.exp(-h1)) * h3
        y_ref[...] = jnp.dot(act, w2_ref[0], preferred_element_type=jnp.float32)


def _moe(xs, w1, w3, w2, gmap, valid):
    grid = (NB,)
    return pl.pallas_call(
        _moe_body,
        grid_spec=pltpu.PrefetchScalarGridSpec(
            num_scalar_prefetch=2,
            grid=grid,
            in_specs=[
                pl.BlockSpec((BM, D), lambda b, g, v: (b, 0)),
                pl.BlockSpec((1, D, FB), lambda b, g, v: (g[b], 0, 0)),
                pl.BlockSpec((1, D, FB), lambda b, g, v: (g[b], 0, 0)),
                pl.BlockSpec((1, FB, D), lambda b, g, v: (g[b], 0, 0)),
            ],
            out_specs=pl.BlockSpec((BM, D), lambda b, g, v: (b, 0)),
        ),
        out_shape=jax.ShapeDtypeStruct((M, D), jnp.float32),
        compiler_params=pltpu.CompilerParams(
            dimension_semantics=("arbitrary",),
            vmem_limit_bytes=128 * 1024 * 1024),
    )(gmap, valid, xs, w1, w3, w2)


_TPW = T // _NW


def _gather2_sc_body(d0_hbm, d1_hbm, y_hbm, ya_hbm, yb_hbm,
                     idx_v, rows_v, sem):
    wid = lax.axis_index("s") * 2 + lax.axis_index("c")
    base = wid * _TPW
    pltpu.sync_copy(d0_hbm.at[pl.ds(base, _TPW)], idx_v)
    pltpu.async_copy(y_hbm.at[idx_v], rows_v, sem).wait()
    pltpu.sync_copy(rows_v, ya_hbm.at[pl.ds(base, _TPW)])
    pltpu.sync_copy(d1_hbm.at[pl.ds(base, _TPW)], idx_v)
    pltpu.async_copy(y_hbm.at[idx_v], rows_v, sem).wait()
    pltpu.sync_copy(rows_v, yb_hbm.at[pl.ds(base, _TPW)])


def _gather2(y, d0, d1):
    mesh = plsc.VectorSubcoreMesh(core_axis_name="c", subcore_axis_name="s")
    return pl.kernel(
        _gather2_sc_body,
        mesh=mesh,
        out_type=[jax.ShapeDtypeStruct((T, D), jnp.float32),
                  jax.ShapeDtypeStruct((T, D), jnp.float32)],
        scratch_types=[
            pltpu.VMEM((_TPW,), jnp.int32),
            pltpu.VMEM((_TPW, D), jnp.float32),
            pltpu.SemaphoreType.DMA,
        ],
    )(d0, d1, y)


def _combine_body(x_ref, ya_ref, yb_ref, w0_ref, w1_ref, out_ref):
    w0 = w0_ref[:, :1]
    w1 = w1_ref[:, :1]
    out_ref[...] = x_ref[...] + w0 * ya_ref[...] + w1 * yb_ref[...]


def _combine(x, ya, yb, w0, w1):
    blk = 256
    grid = (T // blk,)
    spec_x = pl.BlockSpec((blk, D), lambda i: (i, 0))
    spec_w = pl.BlockSpec((blk, 128), lambda i: (i, 0))
    return pl.pallas_call(
        _combine_body,
        grid=grid,
        in_specs=[spec_x, spec_x, spec_x, spec_w, spec_w],
        out_specs=spec_x,
        out_shape=jax.ShapeDtypeStruct((T, D), jnp.float32),
    )(x, ya, yb, w0, w1)


def kernel(hidden_states, ln1_g, ln1_b, Wq, Wk, Wv, Wo, ln2_g, ln2_b,
           Wr, W1, W3, W2):
    b, s, d = hidden_states.shape
    hid = hidden_states.reshape(T, D)

    qh, kh, vh = _qkv(hid, ln1_g, ln1_b, Wq, Wk, Wv)
    ctx = _attn(qh, kh, vh)
    x, h2, lg = _proj(ctx, hid, Wo, ln2_g, ln2_b, Wr)

    d0, d1, w0, w1, gmap, valid, loss = _router(lg)
    d0 = d0.reshape(T)
    d1 = d1.reshape(T)

    src = jnp.concatenate([jnp.arange(T, dtype=jnp.int32)] * 2)
    dst = jnp.concatenate([d0, d1])
    xs = _dispatch(h2, src, dst)

    y = _moe(xs, W1, W3, W2, gmap, valid)

    ya, yb = _gather2(y, d0, d1)
    out = _combine(x, ya, yb, w0, w1).reshape(b, s, d)
    return (out, loss[0, 0])


# speedup vs baseline: 1.0420x; 1.0420x over previous
import functools

import jax
import jax.numpy as jnp
import numpy as np
from jax import lax
from jax.experimental import pallas as pl
from jax.experimental.pallas import tpu as pltpu
from jax.experimental.pallas import tpu_sc as plsc

T = 2048
D = 768
H = 12
DH = D // H
E = 8
FF = 4 * D

BM = 256
NB = 2 * T // BM + E
M = NB * BM
FB = 3072
FT = FF // FB

_NEG = float(np.finfo(np.float32).min)


def _qkv_body(x_ref, g_ref, b_ref, w_ref, q_ref, k_ref, v_ref):
    x = x_ref[...]
    m = jnp.mean(x, axis=-1, keepdims=True)
    v = jnp.mean((x - m) * (x - m), axis=-1, keepdims=True)
    h = (x - m) * jax.lax.rsqrt(v + 1e-5) * g_ref[...] + b_ref[...]
    qkv = jnp.dot(h, w_ref[...], preferred_element_type=jnp.float32)
    for hh in range(H):
        q_ref[hh] = qkv[:, hh * DH:(hh + 1) * DH]
        k_ref[hh] = qkv[:, D + hh * DH:D + (hh + 1) * DH]
        v_ref[hh] = qkv[:, 2 * D + hh * DH:2 * D + (hh + 1) * DH]


def _qkv(x, g, b, wq, wk, wv):
    blk = 256
    grid = (T // blk,)
    wqkv = jnp.concatenate([wq, wk, wv], axis=1)
    spec_x = pl.BlockSpec((blk, D), lambda i: (i, 0))
    spec_w = pl.BlockSpec((D, 3 * D), lambda i: (0, 0))
    spec_v = pl.BlockSpec((1, D), lambda i: (0, 0))
    spec_h = pl.BlockSpec((H, blk, DH), lambda i: (0, i, 0))
    return pl.pallas_call(
        _qkv_body,
        grid=grid,
        in_specs=[spec_x, spec_v, spec_v, spec_w],
        out_specs=[spec_h, spec_h, spec_h],
        out_shape=[jax.ShapeDtypeStruct((H, T, DH), jnp.float32)] * 3,
    )(x, g.reshape(1, D), b.reshape(1, D), wqkv)


def _attn_body(q_ref, k_ref, v_ref, o_ref, *, blk):
    i = pl.program_id(1)
    q = q_ref[0]
    k = k_ref[0]
    s = jax.lax.dot_general(q, k, (((1,), (1,)), ((), ())),
                            preferred_element_type=jnp.float32)
    s = s * (1.0 / np.sqrt(DH))
    rows = jax.lax.broadcasted_iota(jnp.int32, (blk, T), 0) + i * blk
    cols = jax.lax.broadcasted_iota(jnp.int32, (blk, T), 1)
    s = jnp.where(cols <= rows, s, _NEG)
    m = jnp.max(s, axis=-1, keepdims=True)
    p = jnp.exp(s - m)
    l = jnp.sum(p, axis=-1, keepdims=True)
    o = jnp.dot(p, v_ref[0], preferred_element_type=jnp.float32)
    o_ref[0] = o / l


def _attn(q, k, v):
    blk = 1024
    grid = (H, T // blk)
    return pl.pallas_call(
        functools.partial(_attn_body, blk=blk),
        grid=grid,
        in_specs=[
            pl.BlockSpec((1, blk, DH), lambda h, i: (h, i, 0)),
            pl.BlockSpec((1, T, DH), lambda h, i: (h, 0, 0)),
            pl.BlockSpec((1, T, DH), lambda h, i: (h, 0, 0)),
        ],
        out_specs=pl.BlockSpec((1, blk, DH), lambda h, i: (h, i, 0)),
        out_shape=jax.ShapeDtypeStruct((H, T, DH), jnp.float32),
    )(q, k, v)


def _proj_body(ctx_ref, hid_ref, wo_ref, g_ref, b_ref, wr_ref,
               x_ref, h2_ref, lg_ref):
    ctx = jnp.concatenate([ctx_ref[hh] for hh in range(H)], axis=-1)
    x = hid_ref[...] + jnp.dot(ctx, wo_ref[...],
                               preferred_element_type=jnp.float32)
    x_ref[...] = x
    m = jnp.mean(x, axis=-1, keepdims=True)
    v = jnp.mean((x - m) * (x - m), axis=-1, keepdims=True)
    h2 = (x - m) * jax.lax.rsqrt(v + 1e-5) * g_ref[...] + b_ref[...]
    h2_ref[...] = h2
    lg_ref[...] = jnp.dot(h2, wr_ref[...], preferred_element_type=jnp.float32)


def _proj(ctx, hid, wo, g, b, wr):
    blk = 256
    grid = (T // blk,)
    spec_x = pl.BlockSpec((blk, D), lambda i: (i, 0))
    return pl.pallas_call(
        _proj_body,
        grid=grid,
        in_specs=[pl.BlockSpec((H, blk, DH), lambda i: (0, i, 0)), spec_x,
                  pl.BlockSpec((D, D), lambda i: (0, 0)),
                  pl.BlockSpec((1, D), lambda i: (0, 0)),
                  pl.BlockSpec((1, D), lambda i: (0, 0)),
                  pl.BlockSpec((D, E), lambda i: (0, 0))],
        out_specs=[spec_x, spec_x, pl.BlockSpec((blk, E), lambda i: (i, 0))],
        out_shape=[jax.ShapeDtypeStruct((T, D), jnp.float32),
                   jax.ShapeDtypeStruct((T, D), jnp.float32),
                   jax.ShapeDtypeStruct((T, E), jnp.float32)],
    )(ctx, hid, wo, g.reshape(1, D), b.reshape(1, D), wr)


def _cumsum0(x):
    sh = 1
    while sh < T:
        x = x + jnp.concatenate(
            [jnp.zeros((sh, E), x.dtype), x[:-sh, :]], axis=0)
        sh *= 2
    return x


def _router_body(lg_ref, d0_ref, d1_ref, w0_ref, w1_ref,
                 gmap_ref, valid_ref, loss_ref):
    lg = lg_ref[...]
    eio = jax.lax.broadcasted_iota(jnp.int32, (T, E), 1)

    v0 = jnp.max(lg, axis=-1, keepdims=True)
    i0 = jnp.min(jnp.where(lg == v0, eio, E), axis=-1, keepdims=True)
    lg1 = jnp.where(eio == i0, _NEG, lg)
    v1 = jnp.max(lg1, axis=-1, keepdims=True)
    i1 = jnp.min(jnp.where(lg1 == v1, eio, E), axis=-1, keepdims=True)

    e1v = jnp.exp(v1 - v0)
    den = 1.0 + e1v
    w0 = 1.0 / den
    w1 = e1v / den
    w0_ref[...] = jnp.broadcast_to(w0, (T, 128))
    w1_ref[...] = jnp.broadcast_to(w1, (T, 128))

    p = jnp.exp(lg - v0)
    probs = p / jnp.sum(p, axis=-1, keepdims=True)
    usage = jnp.sum(probs, axis=0) * (1.0 / T)
    loss_ref[0, 0] = E * jnp.sum(usage * usage)

    oh0 = (eio == i0).astype(jnp.float32)
    oh1 = (eio == i1).astype(jnp.float32)
    c0in = _cumsum0(oh0)
    c1in = _cumsum0(oh1)
    rank0 = jnp.sum(oh0 * (c0in - oh0), axis=-1)
    rank1 = jnp.sum(oh1 * (c1in - oh1), axis=-1)
    cnt0 = c0in[T - 1, :]
    cnt1 = c1in[T - 1, :]

    off = jnp.float32(0.0)
    offs = []
    ends = []
    for e in range(E):
        ce = cnt0[e] + cnt1[e]
        nb_e = jnp.ceil(ce * (1.0 / BM))
        offs.append(off)
        end = off + nb_e * BM
        ends.append(end)
        off = end

    base0 = jnp.zeros((T,), jnp.float32)
    base1 = jnp.zeros((T,), jnp.float32)
    for e in range(E):
        base0 = base0 + oh0[:, e] * offs[e]
        base1 = base1 + oh1[:, e] * (offs[e] + cnt0[e])
    d0_ref[...] = (base0 + rank0).astype(jnp.int32)[None, :]
    d1_ref[...] = (base1 + rank1).astype(jnp.int32)[None, :]

    for b in range(NB):
        g = jnp.int32(0)
        for e in range(E - 1):
            g = g + (ends[e] <= b * BM).astype(jnp.int32)
        gmap_ref[b] = g
        valid_ref[b] = (b * BM < off).astype(jnp.int32)


def _router(lg):
    vm = pl.BlockSpec(memory_space=pltpu.VMEM)
    sm = pl.BlockSpec(memory_space=pltpu.SMEM)
    return pl.pallas_call(
        _router_body,
        in_specs=[vm],
        out_specs=[vm, vm, vm, vm, sm, sm, sm],
        out_shape=[jax.ShapeDtypeStruct((1, T), jnp.int32),
                   jax.ShapeDtypeStruct((1, T), jnp.int32),
                   jax.ShapeDtypeStruct((T, 128), jnp.float32),
                   jax.ShapeDtypeStruct((T, 128), jnp.float32),
                   jax.ShapeDtypeStruct((NB,), jnp.int32),
                   jax.ShapeDtypeStruct((NB,), jnp.int32),
                   jax.ShapeDtypeStruct((1, 1), jnp.float32)],
    )(lg)


_NW = 32
_PPW = 2 * T // _NW


def _dispatch_sc_body(src_hbm, dst_hbm, h2_hbm, out_hbm,
                      src_v, dst_v, rows_v, sem):
    wid = lax.axis_index("s") * 2 + lax.axis_index("c")
    base = wid * _PPW
    pltpu.sync_copy(src_hbm.at[pl.ds(base, _PPW)], src_v)
    pltpu.sync_copy(dst_hbm.at[pl.ds(base, _PPW)], dst_v)
    pltpu.async_copy(h2_hbm.at[src_v], rows_v, sem).wait()
    pltpu.async_copy(rows_v, out_hbm.at[dst_v], sem).wait()


def _dispatch(h2, src, dst):
    mesh = plsc.VectorSubcoreMesh(core_axis_name="c", subcore_axis_name="s")
    return pl.kernel(
        _dispatch_sc_body,
        mesh=mesh,
        out_type=jax.ShapeDtypeStruct((M, D), jnp.float32),
        scratch_types=[
            pltpu.VMEM((_PPW,), jnp.int32),
            pltpu.VMEM((_PPW,), jnp.int32),
            pltpu.VMEM((_PPW, D), jnp.float32),
            pltpu.SemaphoreType.DMA,
        ],
    )(src, dst, h2)


def _moe_body(gmap_ref, valid_ref, xs_ref, w1_ref, w3_ref, w2_ref, y_ref):
    b = pl.program_id(0)

    @pl.when(valid_ref[b] == 1)
    def _():
        x = xs_ref[...]
        h1 = jnp.dot(x, w1_ref[0], preferred_element_type=jnp.float32)
        h3 = jnp.dot(x, w3_ref[0], preferred_element_type=jnp.float32)
        act = h1 / (1.0 + jnp.exp(-h1)) * h3
        y_ref[...] = jnp.dot(act, w2_ref[0], preferred_element_type=jnp.float32)


def _moe(xs, w1, w3, w2, gmap, valid):
    grid = (NB,)
    return pl.pallas_call(
        _moe_body,
        grid_spec=pltpu.PrefetchScalarGridSpec(
            num_scalar_prefetch=2,
            grid=grid,
            in_specs=[
                pl.BlockSpec((BM, D), lambda b, g, v: (b, 0)),
                pl.BlockSpec((1, D, FB), lambda b, g, v: (g[b], 0, 0)),
                pl.BlockSpec((1, D, FB), lambda b, g, v: (g[b], 0, 0)),
                pl.BlockSpec((1, FB, D), lambda b, g, v: (g[b], 0, 0)),
            ],
            out_specs=pl.BlockSpec((BM, D), lambda b, g, v: (b, 0)),
        ),
        out_shape=jax.ShapeDtypeStruct((M, D), jnp.float32),
        compiler_params=pltpu.CompilerParams(
            dimension_semantics=("arbitrary",),
            vmem_limit_bytes=128 * 1024 * 1024),
    )(gmap, valid, xs, w1, w3, w2)


_TPW = T // _NW


def _gather2_sc_body(d0_hbm, d1_hbm, y_hbm, ya_hbm, yb_hbm,
                     idx_v, rows_v, sem):
    wid = lax.axis_index("s") * 2 + lax.axis_index("c")
    base = wid * _TPW
    pltpu.sync_copy(d0_hbm.at[pl.ds(base, _TPW)], idx_v)
    pltpu.async_copy(y_hbm.at[idx_v], rows_v, sem).wait()
    pltpu.sync_copy(rows_v, ya_hbm.at[pl.ds(base, _TPW)])
    pltpu.sync_copy(d1_hbm.at[pl.ds(base, _TPW)], idx_v)
    pltpu.async_copy(y_hbm.at[idx_v], rows_v, sem).wait()
    pltpu.sync_copy(rows_v, yb_hbm.at[pl.ds(base, _TPW)])


def _gather2(y, d0, d1):
    mesh = plsc.VectorSubcoreMesh(core_axis_name="c", subcore_axis_name="s")
    return pl.kernel(
        _gather2_sc_body,
        mesh=mesh,
        out_type=[jax.ShapeDtypeStruct((T, D), jnp.float32),
                  jax.ShapeDtypeStruct((T, D), jnp.float32)],
        scratch_types=[
            pltpu.VMEM((_TPW,), jnp.int32),
            pltpu.VMEM((_TPW, D), jnp.float32),
            pltpu.SemaphoreType.DMA,
        ],
    )(d0, d1, y)


def _combine_body(x_ref, ya_ref, yb_ref, w0_ref, w1_ref, out_ref):
    w0 = w0_ref[:, :1]
    w1 = w1_ref[:, :1]
    out_ref[...] = x_ref[...] + w0 * ya_ref[...] + w1 * yb_ref[...]


def _combine(x, ya, yb, w0, w1):
    blk = 256
    grid = (T // blk,)
    spec_x = pl.BlockSpec((blk, D), lambda i: (i, 0))
    spec_w = pl.BlockSpec((blk, 128), lambda i: (i, 0))
    return pl.pallas_call(
        _combine_body,
        grid=grid,
        in_specs=[spec_x, spec_x, spec_x, spec_w, spec_w],
        out_specs=spec_x,
        out_shape=jax.ShapeDtypeStruct((T, D), jnp.float32),
    )(x, ya, yb, w0, w1)


def kernel(hidden_states, ln1_g, ln1_b, Wq, Wk, Wv, Wo, ln2_g, ln2_b,
           Wr, W1, W3, W2):
    b, s, d = hidden_states.shape
    hid = hidden_states.reshape(T, D)

    qh, kh, vh = _qkv(hid, ln1_g, ln1_b, Wq, Wk, Wv)
    ctx = _attn(qh, kh, vh)
    x, h2, lg = _proj(ctx, hid, Wo, ln2_g, ln2_b, Wr)

    d0, d1, w0, w1, gmap, valid, loss = _router(lg)
    d0 = d0.reshape(T)
    d1 = d1.reshape(T)

    src = jnp.concatenate([jnp.arange(T, dtype=jnp.int32)] * 2)
    dst = jnp.concatenate([d0, d1])
    xs = _dispatch(h2, src, dst)

    y = _moe(xs, W1, W3, W2, gmap, valid)

    ya, yb = _gather2(y, d0, d1)
    out = _combine(x, ya, yb, w0, w1).reshape(b, s, d)
    return (out, loss[0, 0])
